# baseline (speedup 1.0000x reference)
import jax
import jax.numpy as jnp
from jax import lax
from jax.experimental import pallas as pl
from jax.experimental.pallas import tpu as pltpu
from jax.experimental.pallas import tpu_sc as plsc

B = 16384
TWO_B = 2 * B
N = 100000
D = 64
TD = 32
NC = 2
NS = 16
NW = NC * NS

_MESH = plsc.VectorSubcoreMesh(core_axis_name="c", subcore_axis_name="s")
_SC_PARAMS = pltpu.CompilerParams(use_tc_tiling_on_sc=False)


def _mm(a, b):
  return lax.dot_general(a, b, (((1,), (0,)), ((), ())),
                         preferred_element_type=jnp.float32)


def _outer(a, b):
  return lax.dot_general(a, b, (((0,), (0,)), ((), ())),
                         precision=lax.Precision.HIGHEST,
                         preferred_element_type=jnp.float32)


def _k1_body(ids1d, mem_tbl, last_update, mem_rows, lu3, idx_v, rbuf, lubuf):
  c = lax.axis_index("c")
  s = lax.axis_index("s")
  w = c * NS + s
  pltpu.sync_copy(ids1d.at[pl.ds(w * 1024, 1024)], idx_v)
  pltpu.sync_copy(mem_tbl.at[idx_v], rbuf)
  pltpu.sync_copy(last_update.at[idx_v], lubuf)
  pltpu.sync_copy(rbuf, mem_rows.at[pl.ds(w * 1024, 1024), :])
  pltpu.sync_copy(lubuf, lu3.at[w // 8, 0, pl.ds((w % 8) * 1024, 1024)])


_k1 = pl.kernel(
    _k1_body,
    out_type=(
        jax.ShapeDtypeStruct((TWO_B, D), jnp.float32),
        jax.ShapeDtypeStruct((TWO_B // 8192, 1, 8192), jnp.int32),
    ),
    mesh=_MESH,
    compiler_params=_SC_PARAMS,
    scratch_types=[
        pltpu.VMEM((1024,), jnp.int32),
        pltpu.VMEM((1024, D), jnp.float32),
        pltpu.VMEM((1024,), jnp.int32),
    ],
)


def _k2_body(m1, m2, raw, t3, lu3, wrow, btime, wt, p_out):
  trel = (t3[0] - lu3[0]).astype(jnp.float32)
  tenc = jnp.cos(_outer(trel, wrow[...]) + btime[...])
  x = jnp.concatenate([m1[...], m2[...], raw[...], tenc], axis=1)
  p_out[...] = _mm(x, wt[...])


def _k2(mem_rows, raw_msg, t3, lu3, wrow, btime, wt):
  RB = 8192
  nb = B // RB
  grid = (nb, 2)
  return pl.pallas_call(
      _k2_body,
      grid=grid,
      in_specs=[
          pl.BlockSpec((RB, D), lambda i, h: (h * nb + i, 0)),
          pl.BlockSpec((RB, D), lambda i, h: ((1 - h) * nb + i, 0)),
          pl.BlockSpec((RB, D), lambda i, h: (i, 0)),
          pl.BlockSpec((1, 1, RB), lambda i, h: (i, 0, 0)),
          pl.BlockSpec((1, 1, RB), lambda i, h: (h * nb + i, 0, 0)),
          pl.BlockSpec((1, TD), lambda i, h: (0, 0)),
          pl.BlockSpec((1, TD), lambda i, h: (0, 0)),
          pl.BlockSpec((224, 192), lambda i, h: (0, 0)),
      ],
      out_specs=pl.BlockSpec((RB, 192), lambda i, h: (h * nb + i, 0)),
      out_shape=jax.ShapeDtypeStruct((TWO_B, 192), jnp.float32),
  )(mem_rows, mem_rows, raw_msg, t3, lu3, wrow, btime, wt)


def _k3_body(ids1d, p_parts, ones8_in, zblk, s_out, cnt8,
             table, ids_v, pbuf, obuf, zscat, ones8_v):
  c = lax.axis_index("c")
  s = lax.axis_index("s")
  r0 = s * 2048
  pltpu.sync_copy(ids1d.at[pl.ds(r0, 2048)], ids_v)
  pltpu.sync_copy(zblk, zscat)
  pltpu.sync_copy(ones8_in, ones8_v)

  @pl.when(c == 0)
  def _counts():
    pltpu.sync_copy(zscat, table.at[ids_v])
    plsc.subcore_barrier()
    pltpu.sync_copy(ones8_v, table.at[ids_v], add=True)
    plsc.subcore_barrier()
    pltpu.sync_copy(table.at[ids_v], obuf)
    pltpu.sync_copy(obuf, cnt8.at[pl.ds(r0, 2048), :])
    plsc.subcore_barrier()

  for q in range(12):
    col0 = c * 96 + q * 8
    pltpu.sync_copy(p_parts.at[pl.ds(r0, 2048), pl.ds(col0, 8)], pbuf)
    pltpu.sync_copy(zscat, table.at[ids_v])
    plsc.subcore_barrier()
    pltpu.sync_copy(pbuf, table.at[ids_v], add=True)
    plsc.subcore_barrier()
    pltpu.sync_copy(table.at[ids_v], obuf)
    pltpu.sync_copy(obuf, s_out.at[pl.ds(r0, 2048), pl.ds(col0, 8)])
    plsc.subcore_barrier()


_k3 = pl.kernel(
    _k3_body,
    out_type=(
        jax.ShapeDtypeStruct((TWO_B, 192), jnp.float32),
        jax.ShapeDtypeStruct((TWO_B, 8), jnp.float32),
    ),
    mesh=_MESH,
    compiler_params=_SC_PARAMS,
    scratch_types=[
        pltpu.VMEM_SHARED((N, 8), jnp.float32),
        pltpu.VMEM((2048,), jnp.int32),
        pltpu.VMEM((2048, 8), jnp.float32),
        pltpu.VMEM((2048, 8), jnp.float32),
        pltpu.VMEM((2048, 8), jnp.float32),
        pltpu.VMEM((2048, 8), jnp.float32),
    ],
)


def _k4_body(s_blk, cnt8, mem_blk, wh, bi, bh, h_out):
  inv = 1.0 / cnt8[:, 0:1]
  h = mem_blk[...]
  gi = s_blk[...] * inv + bi[...]
  gh = _mm(h, wh[...]) + bh[...]
  x = gi + gh
  r = 1.0 / (1.0 + jnp.exp(-x[:, 0:64]))
  z = 1.0 / (1.0 + jnp.exp(-x[:, 64:128]))
  n = jnp.tanh(gi[:, 128:192] + r * gh[:, 128:192])
  h_out[...] = (1.0 - z) * n + z * h


def _k4(s_flat, cnt8, mem_rows, wh, bi, bh):
  RB = 8192
  return pl.pallas_call(
      _k4_body,
      grid=(TWO_B // RB,),
      in_specs=[
          pl.BlockSpec((RB, 192), lambda i: (i, 0)),
          pl.BlockSpec((RB, 8), lambda i: (i, 0)),
          pl.BlockSpec((RB, D), lambda i: (i, 0)),
          pl.BlockSpec((D, 192), lambda i: (0, 0)),
          pl.BlockSpec((1, 192), lambda i: (0, 0)),
          pl.BlockSpec((1, 192), lambda i: (0, 0)),
      ],
      out_specs=pl.BlockSpec((RB, D), lambda i: (i, 0)),
      out_shape=jax.ShapeDtypeStruct((TWO_B, D), jnp.float32),
  )(s_flat, cnt8, mem_rows, wh, bi, bh)


def _k5_body(ids1d, h_new, out_tbl, idx_v, hbuf):
  c = lax.axis_index("c")
  s = lax.axis_index("s")
  w = c * NS + s
  pltpu.sync_copy(ids1d.at[pl.ds(w * 1024, 1024)], idx_v)
  pltpu.sync_copy(h_new.at[pl.ds(w * 1024, 1024), :], hbuf)
  pltpu.sync_copy(hbuf, out_tbl.at[idx_v])


_k5 = pl.kernel(
    _k5_body,
    out_type=(),
    mesh=_MESH,
    compiler_params=_SC_PARAMS,
    scratch_types=[
        pltpu.VMEM((1024,), jnp.int32),
        pltpu.VMEM((1024, D), jnp.float32),
    ],
)


def kernel(src, pos_dst, neg_dst, t, raw_msg, memory, last_update,
           W_time, b_time, W_ih, W_hh, b_ih, b_hh):
  del neg_dst
  ids = jnp.concatenate([src, pos_dst]).astype(jnp.int32)
  mem_ref = jax.new_ref(memory)

  mem_rows, lu3 = _k1(ids, mem_ref, last_update.astype(jnp.int32))

  t3 = t.astype(jnp.int32).reshape(B // 8192, 1, 8192)
  wrow = W_time[:, 0].reshape(1, TD)
  btime = b_time.reshape(1, TD)
  WT = W_ih.T
  p_parts = _k2(mem_rows, raw_msg, t3, lu3, wrow, btime, WT)

  ones8 = jnp.ones((2048, 8), jnp.float32)
  zblk = jnp.zeros((2048, 8), jnp.float32)
  s_parts, cnt8 = _k3(ids, p_parts, ones8, zblk)

  WHT = W_hh.T
  h_new = _k4(s_parts, cnt8, mem_rows, WHT,
              b_ih.reshape(1, 192), b_hh.reshape(1, 192))

  _k5(ids, h_new, mem_ref)
  return mem_ref[...]


# speedup vs baseline: 1.0058x; 1.0058x over previous
import jax
import jax.numpy as jnp
from jax import lax
from jax.experimental import pallas as pl
from jax.experimental.pallas import tpu as pltpu
from jax.experimental.pallas import tpu_sc as plsc

B = 16384
TWO_B = 2 * B
N = 100000
D = 64
TD = 32
NC = 2
NS = 16
NW = NC * NS

_MESH = plsc.VectorSubcoreMesh(core_axis_name="c", subcore_axis_name="s")
_SC_PARAMS = pltpu.CompilerParams(use_tc_tiling_on_sc=False)


def _mm(a, b):
  return lax.dot_general(a, b, (((1,), (0,)), ((), ())),
                         preferred_element_type=jnp.float32)


def _outer(a, b):
  return lax.dot_general(a, b, (((0,), (0,)), ((), ())),
                         precision=lax.Precision.HIGHEST,
                         preferred_element_type=jnp.float32)


def _k1_body(ids1d, mem_tbl, last_update, mem_rows, lu3, idx_v, rbuf, lubuf):
  c = lax.axis_index("c")
  s = lax.axis_index("s")
  w = c * NS + s
  pltpu.sync_copy(ids1d.at[pl.ds(w * 1024, 1024)], idx_v)
  pltpu.sync_copy(mem_tbl.at[idx_v], rbuf)
  pltpu.sync_copy(last_update.at[idx_v], lubuf)
  pltpu.sync_copy(rbuf, mem_rows.at[pl.ds(w * 1024, 1024), :])
  pltpu.sync_copy(lubuf, lu3.at[w // 4, 0, pl.ds((w % 4) * 1024, 1024)])


_k1 = pl.kernel(
    _k1_body,
    out_type=(
        jax.ShapeDtypeStruct((TWO_B, D), jnp.float32),
        jax.ShapeDtypeStruct((TWO_B // 4096, 1, 4096), jnp.int32),
    ),
    mesh=_MESH,
    compiler_params=_SC_PARAMS,
    scratch_types=[
        pltpu.VMEM((1024,), jnp.int32),
        pltpu.VMEM((1024, D), jnp.float32),
        pltpu.VMEM((1024,), jnp.int32),
    ],
)


def _k2_body(m1, m2, raw, t3, lu3, wrow, btime, wt, p_out):
  trel = (t3[0] - lu3[0]).astype(jnp.float32)
  tenc = jnp.cos(_outer(trel, wrow[...]) + btime[...])
  x = jnp.concatenate([m1[...], m2[...], raw[...], tenc], axis=1)
  p_out[...] = _mm(x, wt[...])


def _k2(mem_rows, raw_msg, t3, lu3, wrow, btime, wt):
  RB = 4096
  nb = B // RB
  grid = (nb, 2)
  return pl.pallas_call(
      _k2_body,
      grid=grid,
      in_specs=[
          pl.BlockSpec((RB, D), lambda i, h: (h * nb + i, 0)),
          pl.BlockSpec((RB, D), lambda i, h: ((1 - h) * nb + i, 0)),
          pl.BlockSpec((RB, D), lambda i, h: (i, 0)),
          pl.BlockSpec((1, 1, RB), lambda i, h: (i, 0, 0)),
          pl.BlockSpec((1, 1, RB), lambda i, h: (h * nb + i, 0, 0)),
          pl.BlockSpec((1, TD), lambda i, h: (0, 0)),
          pl.BlockSpec((1, TD), lambda i, h: (0, 0)),
          pl.BlockSpec((224, 192), lambda i, h: (0, 0)),
      ],
      out_specs=pl.BlockSpec((RB, 192), lambda i, h: (h * nb + i, 0)),
      out_shape=jax.ShapeDtypeStruct((TWO_B, 192), jnp.float32),
  )(mem_rows, mem_rows, raw_msg, t3, lu3, wrow, btime, wt)


def _k3_body(ids1d, p_parts, ones8_in, zblk, s_out, cnt8,
             table, ids_v, pbuf, obuf, zscat, ones8_v):
  c = lax.axis_index("c")
  s = lax.axis_index("s")
  r0 = s * 2048
  pltpu.sync_copy(ids1d.at[pl.ds(r0, 2048)], ids_v)
  pltpu.sync_copy(zblk, zscat)
  pltpu.sync_copy(ones8_in, ones8_v)

  @pl.when(c == 0)
  def _counts():
    pltpu.sync_copy(zscat, table.at[ids_v])
    plsc.subcore_barrier()
    pltpu.sync_copy(ones8_v, table.at[ids_v], add=True)
    plsc.subcore_barrier()
    pltpu.sync_copy(table.at[ids_v], obuf)
    pltpu.sync_copy(obuf, cnt8.at[pl.ds(r0, 2048), :])
    plsc.subcore_barrier()

  for q in range(12):
    col0 = c * 96 + q * 8
    pltpu.sync_copy(p_parts.at[pl.ds(r0, 2048), pl.ds(col0, 8)], pbuf)
    pltpu.sync_copy(zscat, table.at[ids_v])
    plsc.subcore_barrier()
    pltpu.sync_copy(pbuf, table.at[ids_v], add=True)
    plsc.subcore_barrier()
    pltpu.sync_copy(table.at[ids_v], obuf)
    pltpu.sync_copy(obuf, s_out.at[pl.ds(r0, 2048), pl.ds(col0, 8)])
    plsc.subcore_barrier()


_k3 = pl.kernel(
    _k3_body,
    out_type=(
        jax.ShapeDtypeStruct((TWO_B, 192), jnp.float32),
        jax.ShapeDtypeStruct((TWO_B, 8), jnp.float32),
    ),
    mesh=_MESH,
    compiler_params=_SC_PARAMS,
    scratch_types=[
        pltpu.VMEM_SHARED((N, 8), jnp.float32),
        pltpu.VMEM((2048,), jnp.int32),
        pltpu.VMEM((2048, 8), jnp.float32),
        pltpu.VMEM((2048, 8), jnp.float32),
        pltpu.VMEM((2048, 8), jnp.float32),
        pltpu.VMEM((2048, 8), jnp.float32),
    ],
)


def _k4_body(s_blk, cnt8, mem_blk, wh, bi, bh, h_out):
  inv = 1.0 / cnt8[:, 0:1]
  h = mem_blk[...]
  gi = s_blk[...] * inv + bi[...]
  gh = _mm(h, wh[...]) + bh[...]
  x = gi + gh
  r = 1.0 / (1.0 + jnp.exp(-x[:, 0:64]))
  z = 1.0 / (1.0 + jnp.exp(-x[:, 64:128]))
  n = jnp.tanh(gi[:, 128:192] + r * gh[:, 128:192])
  h_out[...] = (1.0 - z) * n + z * h


def _k4(s_flat, cnt8, mem_rows, wh, bi, bh):
  RB = 4096
  return pl.pallas_call(
      _k4_body,
      grid=(TWO_B // RB,),
      in_specs=[
          pl.BlockSpec((RB, 192), lambda i: (i, 0)),
          pl.BlockSpec((RB, 8), lambda i: (i, 0)),
          pl.BlockSpec((RB, D), lambda i: (i, 0)),
          pl.BlockSpec((D, 192), lambda i: (0, 0)),
          pl.BlockSpec((1, 192), lambda i: (0, 0)),
          pl.BlockSpec((1, 192), lambda i: (0, 0)),
      ],
      out_specs=pl.BlockSpec((RB, D), lambda i: (i, 0)),
      out_shape=jax.ShapeDtypeStruct((TWO_B, D), jnp.float32),
  )(s_flat, cnt8, mem_rows, wh, bi, bh)


def _k5_body(ids1d, h_new, out_tbl, idx_v, hbuf):
  c = lax.axis_index("c")
  s = lax.axis_index("s")
  w = c * NS + s
  pltpu.sync_copy(ids1d.at[pl.ds(w * 1024, 1024)], idx_v)
  pltpu.sync_copy(h_new.at[pl.ds(w * 1024, 1024), :], hbuf)
  pltpu.sync_copy(hbuf, out_tbl.at[idx_v])


_k5 = pl.kernel(
    _k5_body,
    out_type=(),
    mesh=_MESH,
    compiler_params=_SC_PARAMS,
    scratch_types=[
        pltpu.VMEM((1024,), jnp.int32),
        pltpu.VMEM((1024, D), jnp.float32),
    ],
)


def kernel(src, pos_dst, neg_dst, t, raw_msg, memory, last_update,
           W_time, b_time, W_ih, W_hh, b_ih, b_hh):
  del neg_dst
  ids = jnp.concatenate([src, pos_dst]).astype(jnp.int32)
  mem_ref = jax.new_ref(memory)

  mem_rows, lu3 = _k1(ids, mem_ref, last_update.astype(jnp.int32))

  t3 = t.astype(jnp.int32).reshape(B // 4096, 1, 4096)
  wrow = W_time[:, 0].reshape(1, TD)
  btime = b_time.reshape(1, TD)
  WT = W_ih.T
  p_parts = _k2(mem_rows, raw_msg, t3, lu3, wrow, btime, WT)

  ones8 = jnp.ones((2048, 8), jnp.float32)
  zblk = jnp.zeros((2048, 8), jnp.float32)
  s_parts, cnt8 = _k3(ids, p_parts, ones8, zblk)

  WHT = W_hh.T
  h_new = _k4(s_parts, cnt8, mem_rows, WHT,
              b_ih.reshape(1, 192), b_hh.reshape(1, 192))

  _k5(ids, h_new, mem_ref)
  return mem_ref[...]


# speedup vs baseline: 1.0637x; 1.0575x over previous
import jax
import jax.numpy as jnp
from jax import lax
from jax.experimental import pallas as pl
from jax.experimental.pallas import tpu as pltpu
from jax.experimental.pallas import tpu_sc as plsc

B = 16384
TWO_B = 2 * B
N = 100000
D = 64
TD = 32
NC = 2
NS = 16
NW = NC * NS

_MESH = plsc.VectorSubcoreMesh(core_axis_name="c", subcore_axis_name="s")
_SC_PARAMS = pltpu.CompilerParams(use_tc_tiling_on_sc=False)


def _mm(a, b):
  return lax.dot_general(a, b, (((1,), (0,)), ((), ())),
                         preferred_element_type=jnp.float32)


def _outer(a, b):
  return lax.dot_general(a, b, (((0,), (0,)), ((), ())),
                         precision=lax.Precision.HIGHEST,
                         preferred_element_type=jnp.float32)


def _k1_body(ids1d, mem_tbl, last_update, mem_rows, lu3, idx_v, rbuf, lubuf):
  c = lax.axis_index("c")
  s = lax.axis_index("s")
  w = c * NS + s
  pltpu.sync_copy(ids1d.at[pl.ds(w * 1024, 1024)], idx_v)
  pltpu.sync_copy(mem_tbl.at[idx_v], rbuf)
  pltpu.sync_copy(last_update.at[idx_v], lubuf)
  pltpu.sync_copy(rbuf, mem_rows.at[pl.ds(w * 1024, 1024), :])
  pltpu.sync_copy(lubuf, lu3.at[w // 4, 0, pl.ds((w % 4) * 1024, 1024)])


_k1 = pl.kernel(
    _k1_body,
    out_type=(
        jax.ShapeDtypeStruct((TWO_B, D), jnp.float32),
        jax.ShapeDtypeStruct((TWO_B // 4096, 1, 4096), jnp.int32),
    ),
    mesh=_MESH,
    compiler_params=_SC_PARAMS,
    scratch_types=[
        pltpu.VMEM((1024,), jnp.int32),
        pltpu.VMEM((1024, D), jnp.float32),
        pltpu.VMEM((1024,), jnp.int32),
    ],
)


def _k2_body(m1, m2, raw, t3, lu3, wrow, btime, wt, p_out):
  trel = (t3[0] - lu3[0]).astype(jnp.float32)
  tenc = jnp.cos(_outer(trel, wrow[...]) + btime[...])
  x = jnp.concatenate([m1[...], m2[...], raw[...], tenc], axis=1)
  p_out[...] = _mm(x, wt[...])


def _k2(mem_rows, raw_msg, t3, lu3, wrow, btime, wt):
  RB = 4096
  nb = B // RB
  grid = (nb, 2)
  return pl.pallas_call(
      _k2_body,
      grid=grid,
      in_specs=[
          pl.BlockSpec((RB, D), lambda i, h: (h * nb + i, 0)),
          pl.BlockSpec((RB, D), lambda i, h: ((1 - h) * nb + i, 0)),
          pl.BlockSpec((RB, D), lambda i, h: (i, 0)),
          pl.BlockSpec((1, 1, RB), lambda i, h: (i, 0, 0)),
          pl.BlockSpec((1, 1, RB), lambda i, h: (h * nb + i, 0, 0)),
          pl.BlockSpec((1, TD), lambda i, h: (0, 0)),
          pl.BlockSpec((1, TD), lambda i, h: (0, 0)),
          pl.BlockSpec((224, 192), lambda i, h: (0, 0)),
      ],
      out_specs=pl.BlockSpec((RB, 192), lambda i, h: (h * nb + i, 0)),
      out_shape=jax.ShapeDtypeStruct((TWO_B, 192), jnp.float32),
  )(mem_rows, mem_rows, raw_msg, t3, lu3, wrow, btime, wt)


def _k3_body(ids1d, p_parts, ones8_in, zblk, s_out, cnt8,
             table, ids_v, pbuf_a, pbuf_b, obuf, zscat, sem_in, sem_out):
  c = lax.axis_index("c")
  s = lax.axis_index("s")
  r0 = s * 2048
  pltpu.sync_copy(ids1d.at[pl.ds(r0, 2048)], ids_v)
  pltpu.sync_copy(zblk, zscat)
  pltpu.sync_copy(ones8_in, pbuf_b)

  bufs = [pbuf_a, pbuf_b]
  d_in = pltpu.async_copy(
      p_parts.at[pl.ds(r0, 2048), pl.ds(c * 96, 8)], pbuf_a, sem_in)

  @pl.when(c == 0)
  def _counts():
    pltpu.sync_copy(zscat, table.at[ids_v])
    plsc.subcore_barrier()
    pltpu.sync_copy(pbuf_b, table.at[ids_v], add=True)
    plsc.subcore_barrier()
    pltpu.sync_copy(table.at[ids_v], obuf)
    pltpu.sync_copy(obuf, cnt8.at[pl.ds(r0, 2048), :])
    plsc.subcore_barrier()

  d_out = None
  for q in range(12):
    pb = bufs[q % 2]
    pltpu.sync_copy(zscat, table.at[ids_v])
    d_in.wait()
    if d_out is not None:
      d_out.wait()
    plsc.subcore_barrier()
    pltpu.sync_copy(pb, table.at[ids_v], add=True)
    if q + 1 < 12:
      d_in = pltpu.async_copy(
          p_parts.at[pl.ds(r0, 2048), pl.ds(c * 96 + (q + 1) * 8, 8)],
          bufs[(q + 1) % 2], sem_in)
    plsc.subcore_barrier()
    pltpu.sync_copy(table.at[ids_v], obuf)
    plsc.subcore_barrier()
    d_out = pltpu.async_copy(
        obuf, s_out.at[pl.ds(r0, 2048), pl.ds(c * 96 + q * 8, 8)], sem_out)
  d_out.wait()


_k3 = pl.kernel(
    _k3_body,
    out_type=(
        jax.ShapeDtypeStruct((TWO_B, 192), jnp.float32),
        jax.ShapeDtypeStruct((TWO_B, 8), jnp.float32),
    ),
    mesh=_MESH,
    compiler_params=_SC_PARAMS,
    scratch_types=[
        pltpu.VMEM_SHARED((N, 8), jnp.float32),
        pltpu.VMEM((2048,), jnp.int32),
        pltpu.VMEM((2048, 8), jnp.float32),
        pltpu.VMEM((2048, 8), jnp.float32),
        pltpu.VMEM((2048, 8), jnp.float32),
        pltpu.VMEM((2048, 8), jnp.float32),
        pltpu.SemaphoreType.DMA,
        pltpu.SemaphoreType.DMA,
    ],
)


def _k4_body(s_blk, cnt8, mem_blk, wh, bi, bh, h_out):
  inv = 1.0 / cnt8[:, 0:1]
  h = mem_blk[...]
  gi = s_blk[...] * inv + bi[...]
  gh = _mm(h, wh[...]) + bh[...]
  x = gi + gh
  r = 1.0 / (1.0 + jnp.exp(-x[:, 0:64]))
  z = 1.0 / (1.0 + jnp.exp(-x[:, 64:128]))
  n = jnp.tanh(gi[:, 128:192] + r * gh[:, 128:192])
  h_out[...] = (1.0 - z) * n + z * h


def _k4(s_flat, cnt8, mem_rows, wh, bi, bh):
  RB = 4096
  return pl.pallas_call(
      _k4_body,
      grid=(TWO_B // RB,),
      in_specs=[
          pl.BlockSpec((RB, 192), lambda i: (i, 0)),
          pl.BlockSpec((RB, 8), lambda i: (i, 0)),
          pl.BlockSpec((RB, D), lambda i: (i, 0)),
          pl.BlockSpec((D, 192), lambda i: (0, 0)),
          pl.BlockSpec((1, 192), lambda i: (0, 0)),
          pl.BlockSpec((1, 192), lambda i: (0, 0)),
      ],
      out_specs=pl.BlockSpec((RB, D), lambda i: (i, 0)),
      out_shape=jax.ShapeDtypeStruct((TWO_B, D), jnp.float32),
  )(s_flat, cnt8, mem_rows, wh, bi, bh)


def _k5_body(ids1d, h_new, out_tbl, idx_v, hbuf):
  c = lax.axis_index("c")
  s = lax.axis_index("s")
  w = c * NS + s
  pltpu.sync_copy(ids1d.at[pl.ds(w * 1024, 1024)], idx_v)
  pltpu.sync_copy(h_new.at[pl.ds(w * 1024, 1024), :], hbuf)
  pltpu.sync_copy(hbuf, out_tbl.at[idx_v])


_k5 = pl.kernel(
    _k5_body,
    out_type=(),
    mesh=_MESH,
    compiler_params=_SC_PARAMS,
    scratch_types=[
        pltpu.VMEM((1024,), jnp.int32),
        pltpu.VMEM((1024, D), jnp.float32),
    ],
)


def kernel(src, pos_dst, neg_dst, t, raw_msg, memory, last_update,
           W_time, b_time, W_ih, W_hh, b_ih, b_hh):
  del neg_dst
  ids = jnp.concatenate([src, pos_dst]).astype(jnp.int32)
  mem_ref = jax.new_ref(memory)

  mem_rows, lu3 = _k1(ids, mem_ref, last_update.astype(jnp.int32))

  t3 = t.astype(jnp.int32).reshape(B // 4096, 1, 4096)
  wrow = W_time[:, 0].reshape(1, TD)
  btime = b_time.reshape(1, TD)
  WT = W_ih.T
  p_parts = _k2(mem_rows, raw_msg, t3, lu3, wrow, btime, WT)

  ones8 = jnp.ones((2048, 8), jnp.float32)
  zblk = jnp.zeros((2048, 8), jnp.float32)
  s_parts, cnt8 = _k3(ids, p_parts, ones8, zblk)

  WHT = W_hh.T
  h_new = _k4(s_parts, cnt8, mem_rows, WHT,
              b_ih.reshape(1, 192), b_hh.reshape(1, 192))

  _k5(ids, h_new, mem_ref)
  return mem_ref[...]
